# baseline (speedup 1.0000x reference)
import jax
import jax.numpy as jnp
from jax import lax
from jax.experimental import pallas as pl
from jax.experimental.pallas import tpu as pltpu
from jax.experimental.pallas import tpu_sc as plsc

_NUM_BUCKETS = 1000
_D = 64
_DP = 128
_NC, _NS, _L = 2, 16, 16
_NW = _NC * _NS
_C = 128


def _qr_body(idx_hbm, t_hbm, out_hbm,
             idx_all, qi0, qi1, rows0, rows1, ov0, ov1,
             sem_g0, sem_g1, sem_s0, sem_s1):
    wid = lax.axis_index("s") * _NC + lax.axis_index("c")
    n = idx_hbm.shape[0]
    per_w = n // _NW
    n_chunks = per_w // _C
    nb = jnp.full((_L,), _NUM_BUCKETS, jnp.int32)

    qi = (qi0, qi1)
    rows = (rows0, rows1)
    ov = (ov0, ov1)
    sem_g = (sem_g0, sem_g1)
    sem_s = (sem_s0, sem_s1)

    pltpu.sync_copy(idx_hbm.at[pl.ds(wid * per_w, per_w)], idx_all)

    def fire(c, slot):
        off = c * _C
        for i in range(_C // _L):
            s = pl.ds(off + i * _L, _L)
            v = idx_all[s]
            qi[slot][pl.ds(i * _L, _L)] = lax.div(v, nb)
            qi[slot][pl.ds(_C + i * _L, _L)] = lax.rem(v, nb) + nb
        pltpu.async_copy(t_hbm.at[qi[slot]], rows[slot], sem_g[slot])

    fire(0, 0)

    @pl.loop(0, n_chunks, step=2)
    def pipe(c0):
        for b in range(2):
            c = c0 + b
            nslot = 1 - b

            @pl.when(c + 1 < n_chunks)
            def _():
                fire(c + 1, nslot)

            pltpu.make_async_copy(t_hbm.at[qi[b]], rows[b], sem_g[b]).wait()

            @pl.when(c >= 2)
            def _():
                pltpu.make_async_copy(
                    ov[b], out_hbm.at[pl.ds(0, _C)], sem_s[b]).wait()

            @plsc.parallel_loop(0, _C, unroll=4)
            def mul_body(i):
                for j in range(_D // _L):
                    sj = pl.ds(j * _L, _L)
                    ov[b][i, sj] = rows[b][i, sj] * rows[b][_C + i, sj]

            base = wid * per_w + c * _C
            pltpu.async_copy(ov[b], out_hbm.at[pl.ds(base, _C)], sem_s[b])

    pltpu.make_async_copy(ov0, out_hbm.at[pl.ds(0, _C)], sem_s0).wait()
    pltpu.make_async_copy(ov1, out_hbm.at[pl.ds(0, _C)], sem_s1).wait()


def kernel(inputs, q_table, r_table):
    b, f = inputs.shape
    n = b * f
    per_w = n // _NW
    flat_idx = inputs.reshape(n)
    t_stack = jnp.concatenate(
        [jnp.pad(q_table, ((0, 0), (0, _DP - _D))),
         jnp.pad(r_table, ((0, 0), (0, _DP - _D)))], axis=0)
    mesh = plsc.VectorSubcoreMesh(core_axis_name="c", subcore_axis_name="s")
    out_flat = pl.kernel(
        _qr_body,
        mesh=mesh,
        out_type=jax.ShapeDtypeStruct((n, _D), jnp.float32),
        scratch_types=[
            pltpu.VMEM((per_w,), jnp.int32),
            pltpu.VMEM((2 * _C,), jnp.int32),
            pltpu.VMEM((2 * _C,), jnp.int32),
            pltpu.VMEM((2 * _C, _DP), jnp.float32),
            pltpu.VMEM((2 * _C, _DP), jnp.float32),
            pltpu.VMEM((_C, _D), jnp.float32),
            pltpu.VMEM((_C, _D), jnp.float32),
            pltpu.SemaphoreType.DMA,
            pltpu.SemaphoreType.DMA,
            pltpu.SemaphoreType.DMA,
            pltpu.SemaphoreType.DMA,
        ],
    )(flat_idx, t_stack)
    return out_flat.reshape(b, f, _D)


# speedup vs baseline: 1.0098x; 1.0098x over previous
import jax
import jax.numpy as jnp
from jax import lax
from jax.experimental import pallas as pl
from jax.experimental.pallas import tpu as pltpu
from jax.experimental.pallas import tpu_sc as plsc

_NUM_BUCKETS = 1000
_D = 64
_DP = 128
_NC, _NS, _L = 2, 16, 16
_NW = _NC * _NS
_C = 64
_K = 4


def _qr_body(idx_hbm, t_hbm, out_hbm,
             idx_all, qi0, qi1, qi2, qi3, rows0, rows1, rows2, rows3,
             ov0, ov1, ov2, ov3,
             sg0, sg1, sg2, sg3, ss0, ss1, ss2, ss3):
    wid = lax.axis_index("s") * _NC + lax.axis_index("c")
    n = idx_hbm.shape[0]
    per_w = n // _NW
    n_chunks = per_w // _C
    nb = jnp.full((_L,), _NUM_BUCKETS, jnp.int32)

    qi = (qi0, qi1, qi2, qi3)
    rows = (rows0, rows1, rows2, rows3)
    ov = (ov0, ov1, ov2, ov3)
    sem_g = (sg0, sg1, sg2, sg3)
    sem_s = (ss0, ss1, ss2, ss3)

    pltpu.sync_copy(idx_hbm.at[pl.ds(wid * per_w, per_w)], idx_all)

    def fire(c, slot):
        off = c * _C
        for i in range(_C // _L):
            s = pl.ds(off + i * _L, _L)
            v = idx_all[s]
            qi[slot][pl.ds(i * _L, _L)] = lax.div(v, nb)
            qi[slot][pl.ds(_C + i * _L, _L)] = lax.rem(v, nb) + nb
        pltpu.async_copy(t_hbm.at[qi[slot]], rows[slot], sem_g[slot])

    for p in range(_K - 1):
        fire(p, p)

    @pl.loop(0, n_chunks, step=_K)
    def pipe(c0):
        for b in range(_K):
            c = c0 + b

            @pl.when(c + _K - 1 < n_chunks)
            def _():
                fire(c + _K - 1, (b + _K - 1) % _K)

            pltpu.make_async_copy(t_hbm.at[qi[b]], rows[b], sem_g[b]).wait()

            @pl.when(c >= _K)
            def _():
                pltpu.make_async_copy(
                    ov[b], out_hbm.at[pl.ds(0, _C)], sem_s[b]).wait()

            @plsc.parallel_loop(0, _C, unroll=4)
            def mul_body(i):
                for j in range(_D // _L):
                    sj = pl.ds(j * _L, _L)
                    ov[b][i, sj] = rows[b][i, sj] * rows[b][_C + i, sj]

            base = wid * per_w + c * _C
            pltpu.async_copy(ov[b], out_hbm.at[pl.ds(base, _C)], sem_s[b])

    for b in range(_K):
        pltpu.make_async_copy(ov[b], out_hbm.at[pl.ds(0, _C)], sem_s[b]).wait()


def kernel(inputs, q_table, r_table):
    b, f = inputs.shape
    n = b * f
    per_w = n // _NW
    flat_idx = inputs.reshape(n)
    t_stack = jnp.concatenate(
        [jnp.pad(q_table, ((0, 0), (0, _DP - _D))),
         jnp.pad(r_table, ((0, 0), (0, _DP - _D)))], axis=0)
    mesh = plsc.VectorSubcoreMesh(core_axis_name="c", subcore_axis_name="s")
    out_flat = pl.kernel(
        _qr_body,
        mesh=mesh,
        out_type=jax.ShapeDtypeStruct((n, _D), jnp.float32),
        scratch_types=(
            [pltpu.VMEM((per_w,), jnp.int32)]
            + [pltpu.VMEM((2 * _C,), jnp.int32)] * _K
            + [pltpu.VMEM((2 * _C, _DP), jnp.float32)] * _K
            + [pltpu.VMEM((_C, _D), jnp.float32)] * _K
            + [pltpu.SemaphoreType.DMA] * (2 * _K)
        ),
    )(flat_idx, t_stack)
    return out_flat.reshape(b, f, _D)


# speedup vs baseline: 1.2372x; 1.2252x over previous
import jax
import jax.numpy as jnp
from jax import lax
from jax.experimental import pallas as pl
from jax.experimental.pallas import tpu as pltpu
from jax.experimental.pallas import tpu_sc as plsc

_NUM_BUCKETS = 1000
_D = 64
_DP = 128
_NC, _NS, _L = 2, 16, 16
_NW = _NC * _NS
_G = 8
_GS = 2


def _qr_body(idx_hbm, t_hbm, out_hbm,
             ix0, ix1, qi0, qi1, rows0, rows1, ova, ovb,
             sg0, sg1, ssa, ssb):
    wid = lax.axis_index("s") * _NC + lax.axis_index("c")
    n = idx_hbm.shape[0]
    nb_rows = out_hbm.shape[0]
    f = out_hbm.shape[1]
    per_w = n // _NW
    rows_w = nb_rows // _NW
    n_chunks = rows_w // _G
    c_lk = _G * f
    nb = jnp.full((_L,), _NUM_BUCKETS, jnp.int32)

    ix = (ix0, ix1)
    qi = (qi0, qi1)
    rows = (rows0, rows1)
    ov = (ova, ovb)
    sem_g = (sg0, sg1)
    sem_s = (ssa, ssb)

    def fire(c, slot):
        pltpu.sync_copy(idx_hbm.at[pl.ds(wid * per_w + c * c_lk, c_lk)],
                        ix[slot])
        for i in range(c_lk // _L):
            v = ix[slot][pl.ds(i * _L, _L)]
            qi[slot][pl.ds(i * _L, _L)] = lax.div(v, nb)
            qi[slot][pl.ds(c_lk + i * _L, _L)] = lax.rem(v, nb) + nb
        pltpu.async_copy(t_hbm.at[qi[slot]], rows[slot], sem_g[slot])

    fire(0, 0)

    @pl.loop(0, n_chunks, step=2)
    def pipe(c0):
        for b in range(2):
            c = c0 + b

            @pl.when(c + 1 < n_chunks)
            def _():
                fire(c + 1, 1 - b)

            pltpu.make_async_copy(t_hbm.at[qi[b]], rows[b], sem_g[b]).wait()

            base = wid * rows_w + c * _G
            for h in range(_G // _GS):
                o = h % 2

                if h >= 2:
                    pltpu.make_async_copy(
                        ov[o], out_hbm.at[pl.ds(0, _GS)], sem_s[o]).wait()
                else:

                    @pl.when(c >= 1)
                    def _():
                        pltpu.make_async_copy(
                            ov[o], out_hbm.at[pl.ds(0, _GS)],
                            sem_s[o]).wait()

                for gg in range(_GS):
                    g = h * _GS + gg

                    @plsc.parallel_loop(0, f, unroll=2)
                    def mul_body(i):
                        for j in range(_D // _L):
                            sj = pl.ds(j * _L, _L)
                            ov[o][gg, i, sj] = (
                                rows[b][g * f + i, sj]
                                * rows[b][c_lk + g * f + i, sj])

                pltpu.async_copy(
                    ov[o], out_hbm.at[pl.ds(base + h * _GS, _GS)], sem_s[o])

    pltpu.make_async_copy(ova, out_hbm.at[pl.ds(0, _GS)], ssa).wait()
    pltpu.make_async_copy(ovb, out_hbm.at[pl.ds(0, _GS)], ssb).wait()


def kernel(inputs, q_table, r_table):
    b, f = inputs.shape
    n = b * f
    c_lk = _G * f
    flat_idx = inputs.reshape(n)
    t_stack = jnp.concatenate(
        [jnp.pad(q_table, ((0, 0), (0, _DP - _D))),
         jnp.pad(r_table, ((0, 0), (0, _DP - _D)))], axis=0)
    mesh = plsc.VectorSubcoreMesh(core_axis_name="c", subcore_axis_name="s")
    return pl.kernel(
        _qr_body,
        mesh=mesh,
        out_type=jax.ShapeDtypeStruct((b, f, _D), jnp.float32),
        scratch_types=(
            [pltpu.VMEM((c_lk,), jnp.int32)] * 2
            + [pltpu.VMEM((2 * c_lk,), jnp.int32)] * 2
            + [pltpu.VMEM((2 * c_lk, _DP), jnp.float32)] * 2
            + [pltpu.VMEM((_GS, f, _D), jnp.float32)] * 2
            + [pltpu.SemaphoreType.DMA] * 4
        ),
    )(flat_idx, t_stack)


# speedup vs baseline: 1.4581x; 1.1785x over previous
import jax
import jax.numpy as jnp
from jax import lax
from jax.experimental import pallas as pl
from jax.experimental.pallas import tpu as pltpu
from jax.experimental.pallas import tpu_sc as plsc

_NUM_BUCKETS = 1000
_D = 64
_DP = 128
_NC, _NS, _L = 2, 16, 16
_NW = _NC * _NS
_B = 128


def _qr_body(idx_hbm, t_hbm, out_hbm,
             idx_t, qi0, qi1, rows0, rows1, ova, ovb,
             sg0, sg1, ssa, ssb):
    wid = lax.axis_index("s") * _NC + lax.axis_index("c")
    f = idx_hbm.shape[0]
    nb_rows = idx_hbm.shape[1]
    rows_w = nb_rows // _NW
    blks = rows_w // _B
    _BS = blks.bit_length() - 1
    n_chunks = f * blks
    nb = jnp.full((_L,), _NUM_BUCKETS, jnp.int32)

    qi = (qi0, qi1)
    rows = (rows0, rows1)
    ov = (ova, ovb)
    sem_g = (sg0, sg1)
    sem_s = (ssa, ssb)

    pltpu.sync_copy(idx_hbm.at[:, pl.ds(wid * rows_w, rows_w)], idx_t)

    def fire(c, slot):
        fld = lax.shift_right_logical(c, _BS)
        blk = lax.bitwise_and(c, blks - 1)
        for i in range(_B // _L):
            v = idx_t[fld, pl.ds(blk * _B + i * _L, _L)]
            qi[slot][pl.ds(i * _L, _L)] = lax.div(v, nb)
            qi[slot][pl.ds(_B + i * _L, _L)] = lax.rem(v, nb) + nb
        pltpu.async_copy(t_hbm.at[qi[slot]], rows[slot], sem_g[slot])

    fire(0, 0)

    dvecs = [lax.iota(jnp.int32, _L) + jnp.full((_L,), j * _L, jnp.int32)
             for j in range(_D // _L)]

    @pl.loop(0, n_chunks, step=2)
    def pipe(c0):
        for b in range(2):
            c = c0 + b

            @pl.when(c + 1 < n_chunks)
            def _():
                fire(c + 1, 1 - b)

            pltpu.make_async_copy(t_hbm.at[qi[b]], rows[b], sem_g[b]).wait()

            @pl.when(c >= 2)
            def _():
                pltpu.make_async_copy(
                    ov[b], out_hbm.at[0, :, pl.ds(0, _B)], sem_s[b]).wait()

            @plsc.parallel_loop(0, _B, unroll=2)
            def mul_body(l):
                bv = jnp.zeros((_L,), jnp.int32) + l
                for j in range(_D // _L):
                    sj = pl.ds(j * _L, _L)
                    v = rows[b][l, sj] * rows[b][_B + l, sj]
                    plsc.store_scatter(ov[b], [dvecs[j], bv], v)

            fld = lax.shift_right_logical(c, _BS)
            blk = lax.bitwise_and(c, blks - 1)
            pltpu.async_copy(
                ov[b],
                out_hbm.at[fld, :, pl.ds(wid * rows_w + blk * _B, _B)],
                sem_s[b])

    pltpu.make_async_copy(ova, out_hbm.at[0, :, pl.ds(0, _B)], ssa).wait()
    pltpu.make_async_copy(ovb, out_hbm.at[0, :, pl.ds(0, _B)], ssb).wait()


def kernel(inputs, q_table, r_table):
    b, f = inputs.shape
    idx_t = inputs.T
    t_stack = jnp.concatenate(
        [jnp.pad(q_table, ((0, 0), (0, _DP - _D))),
         jnp.pad(r_table, ((0, 0), (0, _DP - _D)))], axis=0)
    mesh = plsc.VectorSubcoreMesh(core_axis_name="c", subcore_axis_name="s")
    out_phys = pl.kernel(
        _qr_body,
        mesh=mesh,
        compiler_params=pltpu.CompilerParams(needs_layout_passes=False),
        out_type=jax.ShapeDtypeStruct((f, _D, b), jnp.float32),
        scratch_types=(
            [pltpu.VMEM((f, b // _NW), jnp.int32)]
            + [pltpu.VMEM((2 * _B,), jnp.int32)] * 2
            + [pltpu.VMEM((2 * _B, _DP), jnp.float32)] * 2
            + [pltpu.VMEM((_D, _B), jnp.float32)] * 2
            + [pltpu.SemaphoreType.DMA] * 4
        ),
    )(idx_t, t_stack)
    return jnp.transpose(out_phys, (2, 0, 1))
